# baseline (speedup 1.0000x reference)
import functools

import numpy as np

import jax
import jax.numpy as jnp
from jax import lax
from jax.experimental import pallas as pl
from jax.experimental.pallas import tpu as pltpu
from jax.experimental.pallas import tpu_sc as plsc

BINS = 30
NBINS_PAD = 64
B, C, Q, D = 32, 8, 32, 512
ROWS = B * C * Q
ROWS_PER_W = C * Q
CHUNK = 16
N_CHUNKS = ROWS_PER_W // CHUNK
CHUNK_ELEMS = CHUNK * D
HIST_PER_CHUNK = CHUNK * NBINS_PAD
LANES = 16
GATHER_WIN = CHUNK_ELEMS - (D // LANES - 1) * LANES
VALID_BIAS = 1.00001
JUNK_BIAS = 3.0

_NC = 2

def _hist_kernel_body(sim_hbm, dtok_hbm, qtok_hbm, out_hbm,
                      buf0, buf1, hist, dtok_v, qtok_v, dbias_v, qmask_v,
                      sem0, sem1):
    wid = lax.axis_index("s") * _NC + lax.axis_index("c")
    sim_base = wid * (ROWS_PER_W * D)

    lane = lax.broadcasted_iota(jnp.int32, (LANES,), 0)
    diag_idx = [lane * D + ((j + lane) & (LANES - 1)) for j in range(LANES)]
    rot1 = (lane + 1) & (LANES - 1)
    lane64 = lane * NBINS_PAD

    def start(k, buf, sem):
        pltpu.make_async_copy(
            sim_hbm.at[pl.ds(sim_base + k * CHUNK_ELEMS, CHUNK_ELEMS)],
            buf, sem).start()

    def wait(buf, sem):
        pltpu.make_async_copy(
            sim_hbm.at[pl.ds(0, CHUNK_ELEMS)], buf, sem).wait()

    start(0, buf0, sem0)
    start(1, buf1, sem1)
    pltpu.sync_copy(dtok_hbm.at[pl.ds(wid * D, D)], dtok_v)
    pltpu.sync_copy(qtok_hbm.at[pl.ds(wid * Q, Q)], qtok_v)

    zeros = jnp.zeros((LANES,), jnp.float32)

    def zbody(i, _):
        for u in range(8):
            hist[pl.ds(i * (8 * LANES) + u * LANES, LANES)] = zeros
        return 0
    lax.fori_loop(0, (ROWS_PER_W * NBINS_PAD) // (8 * LANES), zbody, 0)

    def dbias_body(i, _):
        t = dtok_v[pl.ds(i * LANES, LANES)]
        dbias_v[pl.ds(i * LANES, LANES)] = jnp.where(
            t == -1, jnp.float32(JUNK_BIAS), jnp.float32(VALID_BIAS))
        return 0
    lax.fori_loop(0, D // LANES, dbias_body, 0)

    def qmask_body(i, _):
        t = qtok_v[pl.ds(i * LANES, LANES)]
        qmask_v[pl.ds(i * LANES, LANES)] = jnp.where(
            t == -1, jnp.float32(0.0), jnp.float32(1.0))
        return 0
    lax.fori_loop(0, Q // LANES, qmask_body, 0)

    def compute(k, parity, buf):
        qvals = qmask_v[pl.ds(parity * LANES, LANES)]
        hist_k = hist.at[pl.ds(k * HIST_PER_CHUNK, HIST_PER_CHUNK)]

        def dbody(t, _):
            dmask16 = dbias_v[pl.ds(t * LANES, LANES)]
            sub = buf.at[pl.ds(t * LANES, GATHER_WIN)]
            vs = [plsc.load_gather(sub, [diag_idx[j]])
                  for j in range(LANES)]
            dbcs = [dmask16]
            for _j in range(LANES - 1):
                dbcs.append(jnp.take_along_axis(
                    dbcs[-1], rot1, axis=0, mode="promise_in_bounds"))
            sums = [v + dbc for v, dbc in zip(vs, dbcs)]
            scaled = [s * jnp.float32(14.5) for s in sums]
            bins = [s.astype(jnp.int32) for s in scaled]
            addrs = [bn + lane64 for bn in bins]
            for a in addrs:
                plsc.addupdate_scatter(hist_k, [a], qvals)
            return 0
        lax.fori_loop(0, D // LANES, dbody, 0)

    def pbody(p, _):
        k0 = 2 * p
        wait(buf0, sem0)
        compute(k0, 0, buf0)
        start(k0 + 2, buf0, sem0)
        wait(buf1, sem1)
        compute(k0 + 1, 1, buf1)
        start(k0 + 3, buf1, sem1)
        return 0
    lax.fori_loop(0, N_CHUNKS // 2 - 1, pbody, 0)

    wait(buf0, sem0)
    compute(N_CHUNKS - 2, 0, buf0)
    wait(buf1, sem1)
    compute(N_CHUNKS - 1, 1, buf1)

    pltpu.sync_copy(
        hist, out_hbm.at[pl.ds(wid * ROWS_PER_W * NBINS_PAD,
                               ROWS_PER_W * NBINS_PAD)])


@functools.cache
def _build_kernel():
    mesh = plsc.VectorSubcoreMesh(core_axis_name="c", subcore_axis_name="s")
    return pl.kernel(
        _hist_kernel_body,
        out_type=jax.ShapeDtypeStruct((ROWS * NBINS_PAD,), jnp.float32),
        mesh=mesh,
        compiler_params=pltpu.CompilerParams(needs_layout_passes=False),
        scratch_types=[
            pltpu.VMEM((CHUNK_ELEMS,), jnp.float32),
            pltpu.VMEM((CHUNK_ELEMS,), jnp.float32),
            pltpu.VMEM((ROWS_PER_W * NBINS_PAD,), jnp.float32),
            pltpu.VMEM((D,), jnp.int32),
            pltpu.VMEM((Q,), jnp.int32),
            pltpu.VMEM((D,), jnp.float32),
            pltpu.VMEM((Q,), jnp.float32),
            pltpu.SemaphoreType.DMA,
            pltpu.SemaphoreType.DMA,
        ],
    )


def kernel(simmat, dlens, dtoks, qtoks):
    del dlens
    sim_flat = simmat.reshape(-1)
    dtok_flat = dtoks.astype(jnp.int32).reshape(-1)
    qtok_flat = qtoks.astype(jnp.int32).reshape(-1)
    out = _build_kernel()(sim_flat, dtok_flat, qtok_flat)
    return out.reshape(ROWS, NBINS_PAD)[:, :BINS].reshape(B, C, Q, BINS)


# speedup vs baseline: 1.0217x; 1.0217x over previous
import functools

import numpy as np

import jax
import jax.numpy as jnp
from jax import lax
from jax.experimental import pallas as pl
from jax.experimental.pallas import tpu as pltpu
from jax.experimental.pallas import tpu_sc as plsc

BINS = 30
NBINS_PAD = 64
B, C, Q, D = 32, 8, 32, 512
ROWS = B * C * Q
ROWS_PER_W = C * Q
CHUNK = 16
N_CHUNKS = ROWS_PER_W // CHUNK
CHUNK_ELEMS = CHUNK * D
HIST_PER_CHUNK = CHUNK * NBINS_PAD
LANES = 16
GATHER_WIN = CHUNK_ELEMS - (D // LANES - 1) * LANES
VALID_BIAS = 1.00001
JUNK_BIAS = 3.0

_NC = 2

def _hist_kernel_body(sim_hbm, dtok_hbm, qtok_hbm, out_hbm,
                      buf0, buf1, hist, packed_v, dtok_v, qtok_v,
                      dbias_v, qmask_v, sem0, sem1):
    wid = lax.axis_index("s") * _NC + lax.axis_index("c")
    sim_base = wid * (ROWS_PER_W * D)

    lane = lax.broadcasted_iota(jnp.int32, (LANES,), 0)
    diag_idx = [lane * D + ((j + lane) & (LANES - 1)) for j in range(LANES)]
    rot1 = (lane + 1) & (LANES - 1)

    def start(k, buf, sem):
        pltpu.make_async_copy(
            sim_hbm.at[pl.ds(sim_base + k * CHUNK_ELEMS, CHUNK_ELEMS)],
            buf, sem).start()

    def wait(buf, sem):
        pltpu.make_async_copy(
            sim_hbm.at[pl.ds(0, CHUNK_ELEMS)], buf, sem).wait()

    start(0, buf0, sem0)
    start(1, buf1, sem1)
    pltpu.sync_copy(dtok_hbm.at[pl.ds(wid * D, D)], dtok_v)
    pltpu.sync_copy(qtok_hbm.at[pl.ds(wid * Q, Q)], qtok_v)

    zeros = jnp.zeros((LANES,), jnp.float32)

    def zbody(i, _):
        for u in range(8):
            hist[pl.ds(i * (8 * LANES) + u * LANES, LANES)] = zeros
        return 0
    lax.fori_loop(0, (ROWS_PER_W * NBINS_PAD) // (8 * LANES), zbody, 0)

    def dbias_body(i, _):
        t = dtok_v[pl.ds(i * LANES, LANES)]
        dbias_v[pl.ds(i * LANES, LANES)] = jnp.where(
            t == -1, jnp.float32(JUNK_BIAS), jnp.float32(VALID_BIAS))
        return 0
    lax.fori_loop(0, D // LANES, dbias_body, 0)

    def qmask_body(i, _):
        t = qtok_v[pl.ds(i * LANES, LANES)]
        qmask_v[pl.ds(i * LANES, LANES)] = jnp.where(
            t == -1, jnp.float32(0.0), jnp.float32(1.0))
        return 0
    lax.fori_loop(0, Q // LANES, qmask_body, 0)

    def compute(k, parity, buf):
        qvals = qmask_v[pl.ds(parity * LANES, LANES)]
        hist_k = hist.at[pl.ds(k * HIST_PER_CHUNK, HIST_PER_CHUNK)]

        def dbody(t, _):
            dmask16 = dbias_v[pl.ds(t * LANES, LANES)]
            sub = buf.at[pl.ds(t * LANES, GATHER_WIN)]
            vs = [plsc.load_gather(sub, [diag_idx[j]])
                  for j in range(LANES)]
            dbcs = [dmask16]
            for _j in range(LANES - 1):
                dbcs.append(jnp.take_along_axis(
                    dbcs[-1], rot1, axis=0, mode="promise_in_bounds"))
            sums = [v + dbc for v, dbc in zip(vs, dbcs)]
            scaled = [s * jnp.float32(232.0) for s in sums]
            bins16 = [s.astype(jnp.int32) for s in scaled]
            addrs = [(bn & (-LANES)) | lane for bn in bins16]
            for a in addrs:
                plsc.addupdate_scatter(hist_k, [a], qvals)
            return 0
        lax.fori_loop(0, D // LANES, dbody, 0)

    def pbody(p, _):
        k0 = 2 * p
        wait(buf0, sem0)
        compute(k0, 0, buf0)
        start(k0 + 2, buf0, sem0)
        wait(buf1, sem1)
        compute(k0 + 1, 1, buf1)
        start(k0 + 3, buf1, sem1)
        return 0
    lax.fori_loop(0, N_CHUNKS // 2 - 1, pbody, 0)

    wait(buf0, sem0)
    compute(N_CHUNKS - 2, 0, buf0)
    wait(buf1, sem1)
    compute(N_CHUNKS - 1, 1, buf1)

    tr_idx = [((lane + j) & (LANES - 1)) * LANES + lane for j in range(LANES)]
    out_idx = [lane * BINS + ((lane + j) & (LANES - 1)) for j in range(LANES)]
    hi_mask = [((lane + j) & (LANES - 1)) <= (BINS - LANES - 1)
               for j in range(LANES)]

    def tbody(k, _):
        hist_k = hist.at[pl.ds(k * HIST_PER_CHUNK, HIST_PER_CHUNK)]
        pk = packed_v.at[pl.ds(k * (CHUNK * BINS), CHUNK * BINS)]
        for j in range(LANES):
            g = plsc.load_gather(hist_k, [tr_idx[j]])
            plsc.store_scatter(pk, [out_idx[j]], g)
        for j in range(LANES):
            g = plsc.load_gather(hist_k, [tr_idx[j] + LANES * LANES])
            plsc.store_scatter(pk, [out_idx[j] + LANES], g, mask=hi_mask[j])
        return 0
    lax.fori_loop(0, N_CHUNKS, tbody, 0)

    pltpu.sync_copy(
        packed_v, out_hbm.at[pl.ds(wid * ROWS_PER_W * BINS,
                                   ROWS_PER_W * BINS)])


@functools.cache
def _build_kernel():
    mesh = plsc.VectorSubcoreMesh(core_axis_name="c", subcore_axis_name="s")
    return pl.kernel(
        _hist_kernel_body,
        out_type=jax.ShapeDtypeStruct((ROWS * BINS,), jnp.float32),
        mesh=mesh,
        compiler_params=pltpu.CompilerParams(needs_layout_passes=False),
        scratch_types=[
            pltpu.VMEM((CHUNK_ELEMS,), jnp.float32),
            pltpu.VMEM((CHUNK_ELEMS,), jnp.float32),
            pltpu.VMEM((ROWS_PER_W * NBINS_PAD,), jnp.float32),
            pltpu.VMEM((ROWS_PER_W * BINS,), jnp.float32),
            pltpu.VMEM((D,), jnp.int32),
            pltpu.VMEM((Q,), jnp.int32),
            pltpu.VMEM((D,), jnp.float32),
            pltpu.VMEM((Q,), jnp.float32),
            pltpu.SemaphoreType.DMA,
            pltpu.SemaphoreType.DMA,
        ],
    )


def kernel(simmat, dlens, dtoks, qtoks):
    del dlens
    sim_flat = simmat.reshape(-1)
    dtok_flat = dtoks.astype(jnp.int32).reshape(-1)
    qtok_flat = qtoks.astype(jnp.int32).reshape(-1)
    out = _build_kernel()(sim_flat, dtok_flat, qtok_flat)
    return out.reshape(B, C, Q, BINS)


# speedup vs baseline: 1.2708x; 1.2438x over previous
import functools

import numpy as np

import jax
import jax.numpy as jnp
from jax import lax
from jax.experimental import pallas as pl
from jax.experimental.pallas import tpu as pltpu
from jax.experimental.pallas import tpu_sc as plsc

BINS = 30
NBINS_PAD = 64
B, C, Q, D = 32, 8, 32, 512
ROWS = B * C * Q
ROWS_PER_W = C * Q
CHUNK = 16
N_CHUNKS = ROWS_PER_W // CHUNK
CHUNK_ELEMS = CHUNK * D
HIST_PER_CHUNK = CHUNK * NBINS_PAD
LANES = 16
GATHER_WIN = CHUNK_ELEMS - (D // LANES - 1) * LANES
VALID_BIAS = 1.00001
JUNK_BIAS = 3.0

_NC = 2

def _hist_kernel_body(sim_hbm, dtok_hbm, qtok_hbm, out_hbm,
                      buf0, buf1, hist, packed_v, dtok_v, qtok_v,
                      dbias_v, qmask_v, sem0, sem1):
    wid = lax.axis_index("s") * _NC + lax.axis_index("c")
    sim_base = wid * ROWS_PER_W

    lane = lax.broadcasted_iota(jnp.int32, (LANES,), 0)
    diag_col = [(j + lane) & (LANES - 1) for j in range(LANES)]
    rot1 = (lane + 1) & (LANES - 1)

    def start(k, buf, sem):
        pltpu.make_async_copy(
            sim_hbm.at[pl.ds(sim_base + k * CHUNK, CHUNK), :],
            buf, sem).start()

    def wait(buf, sem):
        pltpu.make_async_copy(
            sim_hbm.at[pl.ds(0, CHUNK), :], buf, sem).wait()

    start(0, buf0, sem0)
    start(1, buf1, sem1)
    pltpu.sync_copy(dtok_hbm.at[pl.ds(wid * D, D)], dtok_v)
    pltpu.sync_copy(qtok_hbm.at[pl.ds(wid * Q, Q)], qtok_v)

    zeros = jnp.zeros((LANES,), jnp.float32)

    def zbody(i, _):
        for u in range(8):
            hist[pl.ds(i * (8 * LANES) + u * LANES, LANES)] = zeros
        return 0
    lax.fori_loop(0, (ROWS_PER_W * NBINS_PAD) // (8 * LANES), zbody, 0)

    def dbias_body(i, _):
        t = dtok_v[pl.ds(i * LANES, LANES)]
        dbias_v[pl.ds(i * LANES, LANES)] = jnp.where(
            t == -1, jnp.float32(JUNK_BIAS), jnp.float32(VALID_BIAS))
        return 0
    lax.fori_loop(0, D // LANES, dbias_body, 0)

    def qmask_body(i, _):
        t = qtok_v[pl.ds(i * LANES, LANES)]
        qmask_v[pl.ds(i * LANES, LANES)] = jnp.where(
            t == -1, jnp.float32(0.0), jnp.float32(1.0))
        return 0
    lax.fori_loop(0, Q // LANES, qmask_body, 0)

    def compute(k, parity, buf):
        qvals = qmask_v[pl.ds(parity * LANES, LANES)]
        hist_k = hist.at[pl.ds(k * HIST_PER_CHUNK, HIST_PER_CHUNK)]

        def dbody(t, _):
            dmask16 = dbias_v[pl.ds(t * LANES, LANES)]
            t16 = t * LANES
            vs = [plsc.load_gather(buf, [lane, diag_col[j] + t16])
                  for j in range(LANES)]
            dbcs = [dmask16]
            for _j in range(LANES - 1):
                dbcs.append(jnp.take_along_axis(
                    dbcs[-1], rot1, axis=0, mode="promise_in_bounds"))
            sums = [v + dbc for v, dbc in zip(vs, dbcs)]
            scaled = [s * jnp.float32(232.0) for s in sums]
            bins16 = [s.astype(jnp.int32) for s in scaled]
            addrs = [(bn & (-LANES)) | lane for bn in bins16]
            for a in addrs:
                plsc.addupdate_scatter(hist_k, [a], qvals)
            return 0
        lax.fori_loop(0, D // LANES, dbody, 0)

    def pbody(p, _):
        k0 = 2 * p
        wait(buf0, sem0)
        compute(k0, 0, buf0)
        start(k0 + 2, buf0, sem0)
        wait(buf1, sem1)
        compute(k0 + 1, 1, buf1)
        start(k0 + 3, buf1, sem1)
        return 0
    lax.fori_loop(0, N_CHUNKS // 2 - 1, pbody, 0)

    wait(buf0, sem0)
    compute(N_CHUNKS - 2, 0, buf0)
    wait(buf1, sem1)
    compute(N_CHUNKS - 1, 1, buf1)

    tr_idx = [((lane + j) & (LANES - 1)) * LANES + lane for j in range(LANES)]
    out_idx = [lane * BINS + ((lane + j) & (LANES - 1)) for j in range(LANES)]
    hi_mask = [((lane + j) & (LANES - 1)) <= (BINS - LANES - 1)
               for j in range(LANES)]

    def tbody(k, _):
        hist_k = hist.at[pl.ds(k * HIST_PER_CHUNK, HIST_PER_CHUNK)]
        pk = packed_v.at[pl.ds(k * (CHUNK * BINS), CHUNK * BINS)]
        for j in range(LANES):
            g = plsc.load_gather(hist_k, [tr_idx[j]])
            plsc.store_scatter(pk, [out_idx[j]], g)
        for j in range(LANES):
            g = plsc.load_gather(hist_k, [tr_idx[j] + LANES * LANES])
            plsc.store_scatter(pk, [out_idx[j] + LANES], g, mask=hi_mask[j])
        return 0
    lax.fori_loop(0, N_CHUNKS, tbody, 0)

    pltpu.sync_copy(
        packed_v, out_hbm.at[pl.ds(wid * ROWS_PER_W * BINS,
                                   ROWS_PER_W * BINS)])


@functools.cache
def _build_kernel():
    mesh = plsc.VectorSubcoreMesh(core_axis_name="c", subcore_axis_name="s")
    return pl.kernel(
        _hist_kernel_body,
        out_type=jax.ShapeDtypeStruct((ROWS * BINS,), jnp.float32),
        mesh=mesh,
        compiler_params=pltpu.CompilerParams(needs_layout_passes=False),
        scratch_types=[
            pltpu.VMEM((CHUNK, D), jnp.float32),
            pltpu.VMEM((CHUNK, D), jnp.float32),
            pltpu.VMEM((ROWS_PER_W * NBINS_PAD,), jnp.float32),
            pltpu.VMEM((ROWS_PER_W * BINS,), jnp.float32),
            pltpu.VMEM((D,), jnp.int32),
            pltpu.VMEM((Q,), jnp.int32),
            pltpu.VMEM((D,), jnp.float32),
            pltpu.VMEM((Q,), jnp.float32),
            pltpu.SemaphoreType.DMA,
            pltpu.SemaphoreType.DMA,
        ],
    )


def kernel(simmat, dlens, dtoks, qtoks):
    del dlens
    sim2d = simmat.reshape(ROWS, D)
    dtok_flat = dtoks.astype(jnp.int32).reshape(-1)
    qtok_flat = qtoks.astype(jnp.int32).reshape(-1)
    out = _build_kernel()(sim2d, dtok_flat, qtok_flat)
    return out.reshape(B, C, Q, BINS)


# speedup vs baseline: 1.3403x; 1.0548x over previous
import functools

import numpy as np

import jax
import jax.numpy as jnp
from jax import lax
from jax.experimental import pallas as pl
from jax.experimental.pallas import tpu as pltpu
from jax.experimental.pallas import tpu_sc as plsc

BINS = 30
NBINS_PAD = 64
B, C, Q, D = 32, 8, 32, 512
ROWS = B * C * Q
ROWS_PER_W = C * Q
CHUNK = 16
N_CHUNKS = ROWS_PER_W // CHUNK
CHUNK_ELEMS = CHUNK * D
HIST_PER_CHUNK = CHUNK * NBINS_PAD
LANES = 16
GATHER_WIN = CHUNK_ELEMS - (D // LANES - 1) * LANES
VALID_BIAS = 1.00001
JUNK_BIAS = 3.0

_NC = 2

def _hist_kernel_body(sim_hbm, dtok_hbm, qtok_hbm, out_hbm,
                      buf0, buf1, hist, packed_v, dtok_v, qtok_v,
                      dbias_v, qmask_v, sem0, sem1):
    wid = lax.axis_index("s") * _NC + lax.axis_index("c")
    sim_base = wid * ROWS_PER_W

    lane = lax.broadcasted_iota(jnp.int32, (LANES,), 0)
    diag_col = [(j + lane) & (LANES - 1) for j in range(LANES)]
    rot1 = (lane + 1) & (LANES - 1)

    def start(k, buf, sem):
        pltpu.make_async_copy(
            sim_hbm.at[pl.ds(sim_base + k * CHUNK, CHUNK), :],
            buf, sem).start()

    def wait(buf, sem):
        pltpu.make_async_copy(
            sim_hbm.at[pl.ds(0, CHUNK), :], buf, sem).wait()

    start(0, buf0, sem0)
    start(1, buf1, sem1)
    pltpu.sync_copy(dtok_hbm.at[pl.ds(wid * D, D)], dtok_v)
    pltpu.sync_copy(qtok_hbm.at[pl.ds(wid * Q, Q)], qtok_v)

    zeros = jnp.zeros((LANES,), jnp.float32)

    def zbody(i, _):
        for u in range(8):
            hist[pl.ds(i * (8 * LANES) + u * LANES, LANES)] = zeros
        return 0
    lax.fori_loop(0, (ROWS_PER_W * NBINS_PAD) // (8 * LANES), zbody, 0)

    def dbias_body(i, _):
        t = dtok_v[pl.ds(i * LANES, LANES)]
        dbias_v[pl.ds(i * LANES, LANES)] = jnp.where(
            t == -1, jnp.float32(JUNK_BIAS), jnp.float32(VALID_BIAS))
        return 0
    lax.fori_loop(0, D // LANES, dbias_body, 0)

    def qmask_body(i, _):
        t = qtok_v[pl.ds(i * LANES, LANES)]
        qmask_v[pl.ds(i * LANES, LANES)] = jnp.where(
            t == -1, jnp.float32(0.0), jnp.float32(1.0))
        return 0
    lax.fori_loop(0, Q // LANES, qmask_body, 0)

    def compute(k, parity, buf):
        qvals = qmask_v[pl.ds(parity * LANES, LANES)]
        hist_k = hist.at[pl.ds(k * HIST_PER_CHUNK, HIST_PER_CHUNK)]

        def dbody(t, _):
            dmask16 = dbias_v[pl.ds(t * LANES, LANES)]
            t16 = t * LANES
            vs = [plsc.load_gather(buf, [lane, diag_col[j] + t16])
                  for j in range(LANES)]
            dbcs = [dmask16]
            for _j in range(LANES - 1):
                dbcs.append(jnp.take_along_axis(
                    dbcs[-1], rot1, axis=0, mode="promise_in_bounds"))
            sums = [v + dbc for v, dbc in zip(vs, dbcs)]
            scaled = [s * jnp.float32(232.0) for s in sums]
            bins16 = [s.astype(jnp.int32) for s in scaled]
            addrs = [(bn & (-LANES)) | lane for bn in bins16]
            for a in addrs:
                plsc.addupdate_scatter(hist_k, [a], qvals)
            return 0
        lax.fori_loop(0, D // LANES, dbody, 0)

    def pbody(p, _):
        k0 = 2 * p
        wait(buf0, sem0)
        compute(k0, 0, buf0)
        start(k0 + 2, buf0, sem0)
        wait(buf1, sem1)
        compute(k0 + 1, 1, buf1)
        start(k0 + 3, buf1, sem1)
        return 0
    lax.fori_loop(0, N_CHUNKS // 2 - 1, pbody, 0)

    wait(buf0, sem0)
    compute(N_CHUNKS - 2, 0, buf0)
    wait(buf1, sem1)
    compute(N_CHUNKS - 1, 1, buf1)

    tr_idx = [((lane + j) & (LANES - 1)) * LANES + lane for j in range(LANES)]
    out_col = [(lane + j) & (LANES - 1) for j in range(LANES)]
    hi_mask = [((lane + j) & (LANES - 1)) <= (BINS - LANES - 1)
               for j in range(LANES)]

    def tbody(k, _):
        hist_k = hist.at[pl.ds(k * HIST_PER_CHUNK, HIST_PER_CHUNK)]
        pk = packed_v.at[pl.ds(k * CHUNK, CHUNK), :]
        for j in range(LANES):
            g = plsc.load_gather(hist_k, [tr_idx[j]])
            plsc.store_scatter(pk, [lane, out_col[j]], g)
        for j in range(LANES):
            g = plsc.load_gather(hist_k, [tr_idx[j] + LANES * LANES])
            plsc.store_scatter(pk, [lane, out_col[j] + LANES], g,
                               mask=hi_mask[j])
        return 0
    lax.fori_loop(0, N_CHUNKS, tbody, 0)

    pltpu.sync_copy(
        packed_v, out_hbm.at[pl.ds(wid * ROWS_PER_W, ROWS_PER_W), :])


@functools.cache
def _build_kernel():
    mesh = plsc.VectorSubcoreMesh(core_axis_name="c", subcore_axis_name="s")
    return pl.kernel(
        _hist_kernel_body,
        out_type=jax.ShapeDtypeStruct((ROWS, BINS), jnp.float32),
        mesh=mesh,
        compiler_params=pltpu.CompilerParams(needs_layout_passes=False),
        scratch_types=[
            pltpu.VMEM((CHUNK, D), jnp.float32),
            pltpu.VMEM((CHUNK, D), jnp.float32),
            pltpu.VMEM((ROWS_PER_W * NBINS_PAD,), jnp.float32),
            pltpu.VMEM((ROWS_PER_W, BINS), jnp.float32),
            pltpu.VMEM((D,), jnp.int32),
            pltpu.VMEM((Q,), jnp.int32),
            pltpu.VMEM((D,), jnp.float32),
            pltpu.VMEM((Q,), jnp.float32),
            pltpu.SemaphoreType.DMA,
            pltpu.SemaphoreType.DMA,
        ],
    )


def kernel(simmat, dlens, dtoks, qtoks):
    del dlens
    sim2d = simmat.reshape(ROWS, D)
    dtok_flat = dtoks.astype(jnp.int32).reshape(-1)
    qtok_flat = qtoks.astype(jnp.int32).reshape(-1)
    out = _build_kernel()(sim2d, dtok_flat, qtok_flat)
    return out.reshape(B, C, Q, BINS)


# speedup vs baseline: 1.3545x; 1.0106x over previous
import functools

import numpy as np

import jax
import jax.numpy as jnp
from jax import lax
from jax.experimental import pallas as pl
from jax.experimental.pallas import tpu as pltpu
from jax.experimental.pallas import tpu_sc as plsc

BINS = 30
NBINS_PAD = 64
B, C, Q, D = 32, 8, 32, 512
ROWS = B * C * Q
ROWS_PER_W = C * Q
CHUNK = 16
N_CHUNKS = ROWS_PER_W // CHUNK
CHUNK_ELEMS = CHUNK * D
HIST_PER_CHUNK = CHUNK * NBINS_PAD
LANES = 16
GATHER_WIN = CHUNK_ELEMS - (D // LANES - 1) * LANES
VALID_BIAS = 1.00001
JUNK_BIAS = 3.0

_NC = 2

def _hist_kernel_body(sim_hbm, dtok_hbm, qtok_hbm, out_hbm,
                      buf0, buf1, hist, packed_v, dtok_v, qtok_v,
                      dbias_v, qmask_v, sem0, sem1):
    wid = lax.axis_index("s") * _NC + lax.axis_index("c")
    sim_base = wid * ROWS_PER_W

    lane = lax.broadcasted_iota(jnp.int32, (LANES,), 0)
    diag_col = [(j + lane) & (LANES - 1) for j in range(LANES)]
    rot1 = (lane + 1) & (LANES - 1)

    def start(k, buf, sem):
        pltpu.make_async_copy(
            sim_hbm.at[pl.ds(sim_base + k * CHUNK, CHUNK), :],
            buf, sem).start()

    def wait(buf, sem):
        pltpu.make_async_copy(
            sim_hbm.at[pl.ds(0, CHUNK), :], buf, sem).wait()

    start(0, buf0, sem0)
    start(1, buf1, sem1)
    pltpu.sync_copy(dtok_hbm.at[wid, :], dtok_v)
    pltpu.sync_copy(qtok_hbm.at[wid, :], qtok_v)

    zeros = jnp.zeros((LANES,), jnp.float32)

    def zbody(i, _):
        for u in range(8):
            hist[pl.ds(i * (8 * LANES) + u * LANES, LANES)] = zeros
        return 0
    lax.fori_loop(0, (ROWS_PER_W * NBINS_PAD) // (8 * LANES), zbody, 0)

    def dbias_body(i, _):
        t = dtok_v[pl.ds(i * LANES, LANES)]
        dbias_v[pl.ds(i * LANES, LANES)] = jnp.where(
            t == -1, jnp.float32(JUNK_BIAS), jnp.float32(VALID_BIAS))
        return 0
    lax.fori_loop(0, D // LANES, dbias_body, 0)

    def qmask_body(i, _):
        t = qtok_v[pl.ds(i * LANES, LANES)]
        qmask_v[pl.ds(i * LANES, LANES)] = jnp.where(
            t == -1, jnp.float32(0.0), jnp.float32(1.0))
        return 0
    lax.fori_loop(0, Q // LANES, qmask_body, 0)

    def compute(k, parity, buf):
        qvals = qmask_v[pl.ds(parity * LANES, LANES)]
        hist_k = hist.at[pl.ds(k * HIST_PER_CHUNK, HIST_PER_CHUNK)]

        def dbody(t, _):
            dmask16 = dbias_v[pl.ds(t * LANES, LANES)]
            t16 = t * LANES
            vs = [plsc.load_gather(buf, [lane, diag_col[j] + t16])
                  for j in range(LANES)]
            dbcs = [dmask16]
            for _j in range(LANES - 1):
                dbcs.append(jnp.take_along_axis(
                    dbcs[-1], rot1, axis=0, mode="promise_in_bounds"))
            sums = [v + dbc for v, dbc in zip(vs, dbcs)]
            scaled = [s * jnp.float32(232.0) for s in sums]
            bins16 = [s.astype(jnp.int32) for s in scaled]
            addrs = [(bn & (-LANES)) | lane for bn in bins16]
            for a in addrs:
                plsc.addupdate_scatter(hist_k, [a], qvals)
            return 0
        lax.fori_loop(0, D // LANES, dbody, 0)

    def pbody(p, _):
        k0 = 2 * p
        more = p < N_CHUNKS // 2 - 1
        wait(buf0, sem0)
        compute(k0, 0, buf0)
        @pl.when(more)
        def _():
            start(k0 + 2, buf0, sem0)
        wait(buf1, sem1)
        compute(k0 + 1, 1, buf1)
        @pl.when(more)
        def _():
            start(k0 + 3, buf1, sem1)
        return 0
    lax.fori_loop(0, N_CHUNKS // 2, pbody, 0)

    tr_idx = [((lane + j) & (LANES - 1)) * LANES + lane for j in range(LANES)]
    out_col = [(lane + j) & (LANES - 1) for j in range(LANES)]
    hi_mask = [((lane + j) & (LANES - 1)) <= (BINS - LANES - 1)
               for j in range(LANES)]

    def tbody(k, _):
        hist_k = hist.at[pl.ds(k * HIST_PER_CHUNK, HIST_PER_CHUNK)]
        pk = packed_v.at[k >> 1, pl.ds((k & 1) * CHUNK, CHUNK), :]
        for j in range(LANES):
            g = plsc.load_gather(hist_k, [tr_idx[j]])
            plsc.store_scatter(pk, [lane, out_col[j]], g)
        for j in range(LANES):
            g = plsc.load_gather(hist_k, [tr_idx[j] + LANES * LANES])
            plsc.store_scatter(pk, [lane, out_col[j] + LANES], g,
                               mask=hi_mask[j])
        return 0
    lax.fori_loop(0, N_CHUNKS, tbody, 0)

    pltpu.sync_copy(packed_v, out_hbm.at[wid])


@functools.cache
def _build_kernel():
    mesh = plsc.VectorSubcoreMesh(core_axis_name="c", subcore_axis_name="s")
    return pl.kernel(
        _hist_kernel_body,
        out_type=jax.ShapeDtypeStruct((B, C, Q, BINS), jnp.float32),
        mesh=mesh,
        compiler_params=pltpu.CompilerParams(needs_layout_passes=False),
        scratch_types=[
            pltpu.VMEM((CHUNK, D), jnp.float32),
            pltpu.VMEM((CHUNK, D), jnp.float32),
            pltpu.VMEM((ROWS_PER_W * NBINS_PAD,), jnp.float32),
            pltpu.VMEM((C, Q, BINS), jnp.float32),
            pltpu.VMEM((D,), jnp.int32),
            pltpu.VMEM((Q,), jnp.int32),
            pltpu.VMEM((D,), jnp.float32),
            pltpu.VMEM((Q,), jnp.float32),
            pltpu.SemaphoreType.DMA,
            pltpu.SemaphoreType.DMA,
        ],
    )


def kernel(simmat, dlens, dtoks, qtoks):
    del dlens
    sim2d = simmat.reshape(ROWS, D)
    return _build_kernel()(sim2d, dtoks.astype(jnp.int32),
                           qtoks.astype(jnp.int32))


# speedup vs baseline: 1.3566x; 1.0015x over previous
import functools

import numpy as np

import jax
import jax.numpy as jnp
from jax import lax
from jax.experimental import pallas as pl
from jax.experimental.pallas import tpu as pltpu
from jax.experimental.pallas import tpu_sc as plsc

BINS = 30
NBINS_PAD = 64
B, C, Q, D = 32, 8, 32, 512
ROWS = B * C * Q
ROWS_PER_W = C * Q
CHUNK = 16
N_CHUNKS = ROWS_PER_W // CHUNK
CHUNK_ELEMS = CHUNK * D
HIST_PER_CHUNK = CHUNK * NBINS_PAD
LANES = 16
GATHER_WIN = CHUNK_ELEMS - (D // LANES - 1) * LANES
VALID_BIAS = 1.00001
JUNK_BIAS = 3.0

_NC = 2

def _hist_kernel_body(sim_hbm, dtok_hbm, qtok_hbm, out_hbm,
                      buf0, buf1, hist, packed_v, dtok_v, qtok_v,
                      dbias_v, qmask_v, sem0, sem1):
    wid = lax.axis_index("s") * _NC + lax.axis_index("c")
    sim_base = wid * ROWS_PER_W

    lane = lax.broadcasted_iota(jnp.int32, (LANES,), 0)
    diag_col = [(j + lane) & (LANES - 1) for j in range(LANES)]
    rot1 = (lane + 1) & (LANES - 1)

    def start(k, buf, sem):
        pltpu.make_async_copy(
            sim_hbm.at[pl.ds(sim_base + k * CHUNK, CHUNK), :],
            buf, sem).start()

    def wait(buf, sem):
        pltpu.make_async_copy(
            sim_hbm.at[pl.ds(0, CHUNK), :], buf, sem).wait()

    start(0, buf0, sem0)
    start(1, buf1, sem1)
    pltpu.sync_copy(dtok_hbm.at[wid, :], dtok_v)
    pltpu.sync_copy(qtok_hbm.at[wid, :], qtok_v)

    zeros = jnp.zeros((LANES,), jnp.float32)

    def zbody(i, _):
        for u in range(BINS):
            hist[pl.ds(i * HIST_PER_CHUNK + u * LANES, LANES)] = zeros
        return 0
    lax.fori_loop(0, N_CHUNKS, zbody, 0)

    def dbias_body(i, _):
        t = dtok_v[pl.ds(i * LANES, LANES)]
        dbias_v[pl.ds(i * LANES, LANES)] = jnp.where(
            t == -1, jnp.float32(JUNK_BIAS), jnp.float32(VALID_BIAS))
        return 0
    lax.fori_loop(0, D // LANES, dbias_body, 0)

    def qmask_body(i, _):
        t = qtok_v[pl.ds(i * LANES, LANES)]
        qmask_v[pl.ds(i * LANES, LANES)] = jnp.where(
            t == -1, jnp.float32(0.0), jnp.float32(1.0))
        return 0
    lax.fori_loop(0, Q // LANES, qmask_body, 0)

    def compute(k, parity, buf):
        qvals = qmask_v[pl.ds(parity * LANES, LANES)]
        hist_k = hist.at[pl.ds(k * HIST_PER_CHUNK, HIST_PER_CHUNK)]

        def dbody(t, _):
            dmask16 = dbias_v[pl.ds(t * LANES, LANES)]
            t16 = t * LANES
            vs = [plsc.load_gather(buf, [lane, diag_col[j] + t16])
                  for j in range(LANES)]
            dbcs = [dmask16]
            for _j in range(LANES - 1):
                dbcs.append(jnp.take_along_axis(
                    dbcs[-1], rot1, axis=0, mode="promise_in_bounds"))
            sums = [v + dbc for v, dbc in zip(vs, dbcs)]
            scaled = [s * jnp.float32(232.0) for s in sums]
            bins16 = [s.astype(jnp.int32) for s in scaled]
            addrs = [(bn & (-LANES)) | lane for bn in bins16]
            for a in addrs:
                plsc.addupdate_scatter(hist_k, [a], qvals)
            return 0
        lax.fori_loop(0, D // LANES, dbody, 0)

    def pbody(p, _):
        k0 = 2 * p
        more = p < N_CHUNKS // 2 - 1
        wait(buf0, sem0)
        compute(k0, 0, buf0)
        @pl.when(more)
        def _():
            start(k0 + 2, buf0, sem0)
        wait(buf1, sem1)
        compute(k0 + 1, 1, buf1)
        @pl.when(more)
        def _():
            start(k0 + 3, buf1, sem1)
        return 0
    lax.fori_loop(0, N_CHUNKS // 2, pbody, 0)

    tr_idx = [((lane + j) & (LANES - 1)) * LANES + lane for j in range(LANES)]
    out_col = [(lane + j) & (LANES - 1) for j in range(LANES)]
    hi_mask = [((lane + j) & (LANES - 1)) <= (BINS - LANES - 1)
               for j in range(LANES)]

    def tbody(k, _):
        hist_k = hist.at[pl.ds(k * HIST_PER_CHUNK, HIST_PER_CHUNK)]
        pk = packed_v.at[k >> 1, pl.ds((k & 1) * CHUNK, CHUNK), :]
        for j in range(LANES):
            g = plsc.load_gather(hist_k, [tr_idx[j]])
            plsc.store_scatter(pk, [lane, out_col[j]], g)
        for j in range(LANES):
            g = plsc.load_gather(hist_k, [tr_idx[j] + LANES * LANES])
            plsc.store_scatter(pk, [lane, out_col[j] + LANES], g,
                               mask=hi_mask[j])
        return 0
    lax.fori_loop(0, N_CHUNKS, tbody, 0)

    pltpu.sync_copy(packed_v, out_hbm.at[wid])


@functools.cache
def _build_kernel():
    mesh = plsc.VectorSubcoreMesh(core_axis_name="c", subcore_axis_name="s")
    return pl.kernel(
        _hist_kernel_body,
        out_type=jax.ShapeDtypeStruct((B, C, Q, BINS), jnp.float32),
        mesh=mesh,
        compiler_params=pltpu.CompilerParams(needs_layout_passes=False),
        scratch_types=[
            pltpu.VMEM((CHUNK, D), jnp.float32),
            pltpu.VMEM((CHUNK, D), jnp.float32),
            pltpu.VMEM((ROWS_PER_W * NBINS_PAD,), jnp.float32),
            pltpu.VMEM((C, Q, BINS), jnp.float32),
            pltpu.VMEM((D,), jnp.int32),
            pltpu.VMEM((Q,), jnp.int32),
            pltpu.VMEM((D,), jnp.float32),
            pltpu.VMEM((Q,), jnp.float32),
            pltpu.SemaphoreType.DMA,
            pltpu.SemaphoreType.DMA,
        ],
    )


def kernel(simmat, dlens, dtoks, qtoks):
    del dlens
    sim2d = simmat.reshape(ROWS, D)
    return _build_kernel()(sim2d, dtoks.astype(jnp.int32),
                           qtoks.astype(jnp.int32))


# speedup vs baseline: 1.3598x; 1.0024x over previous
import functools

import numpy as np

import jax
import jax.numpy as jnp
from jax import lax
from jax.experimental import pallas as pl
from jax.experimental.pallas import tpu as pltpu
from jax.experimental.pallas import tpu_sc as plsc

BINS = 30
NBINS_PAD = 64
B, C, Q, D = 32, 8, 32, 512
ROWS = B * C * Q
ROWS_PER_W = C * Q
CHUNK = 16
N_CHUNKS = ROWS_PER_W // CHUNK
CHUNK_ELEMS = CHUNK * D
HIST_PER_CHUNK = CHUNK * NBINS_PAD
LANES = 16
GATHER_WIN = CHUNK_ELEMS - (D // LANES - 1) * LANES
VALID_BIAS = 1.00001
JUNK_BIAS = 3.0

_NC = 2

def _hist_kernel_body(sim_hbm, dtok_hbm, qtok_hbm, out_hbm,
                      bufs, hist, packed_v, dtok_v, qtok_v,
                      dbias_v, qmask_v, sems):
    wid = lax.axis_index("s") * _NC + lax.axis_index("c")
    sim_base = wid * ROWS_PER_W

    lane = lax.broadcasted_iota(jnp.int32, (LANES,), 0)
    diag_col = [(j + lane) & (LANES - 1) for j in range(LANES)]
    rot1 = (lane + 1) & (LANES - 1)

    def start(k):
        pltpu.make_async_copy(
            sim_hbm.at[pl.ds(sim_base + k * CHUNK, CHUNK), :],
            bufs.at[k & 1], sems.at[k & 1]).start()

    def wait(k):
        pltpu.make_async_copy(
            sim_hbm.at[pl.ds(0, CHUNK), :],
            bufs.at[k & 1], sems.at[k & 1]).wait()

    start(0)
    start(1)
    pltpu.sync_copy(dtok_hbm.at[wid, :], dtok_v)
    pltpu.sync_copy(qtok_hbm.at[wid, :], qtok_v)

    zeros = jnp.zeros((LANES,), jnp.float32)

    def zbody(i, _):
        for u in range(BINS):
            hist[pl.ds(i * HIST_PER_CHUNK + u * LANES, LANES)] = zeros
        return 0
    lax.fori_loop(0, N_CHUNKS, zbody, 0)

    def dbias_body(i, _):
        t = dtok_v[pl.ds(i * LANES, LANES)]
        dbias_v[pl.ds(i * LANES, LANES)] = jnp.where(
            t == -1, jnp.float32(JUNK_BIAS), jnp.float32(VALID_BIAS))
        return 0
    lax.fori_loop(0, D // LANES, dbias_body, 0)

    def qmask_body(i, _):
        t = qtok_v[pl.ds(i * LANES, LANES)]
        qmask_v[pl.ds(i * LANES, LANES)] = jnp.where(
            t == -1, jnp.float32(0.0), jnp.float32(1.0))
        return 0
    lax.fori_loop(0, Q // LANES, qmask_body, 0)

    def compute(k):
        buf = bufs.at[k & 1]
        qvals = qmask_v[pl.ds((k & 1) * LANES, LANES)]
        hist_k = hist.at[pl.ds(k * HIST_PER_CHUNK, HIST_PER_CHUNK)]

        def dbody(t, _):
            dmask16 = dbias_v[pl.ds(t * LANES, LANES)]
            t16 = t * LANES
            vs = [plsc.load_gather(buf, [lane, diag_col[j] + t16])
                  for j in range(LANES)]
            dbcs = [dmask16]
            for _j in range(LANES - 1):
                dbcs.append(jnp.take_along_axis(
                    dbcs[-1], rot1, axis=0, mode="promise_in_bounds"))
            sums = [v + dbc for v, dbc in zip(vs, dbcs)]
            scaled = [s * jnp.float32(232.0) for s in sums]
            bins16 = [s.astype(jnp.int32) for s in scaled]
            addrs = [(bn & (-LANES)) | lane for bn in bins16]
            for a in addrs:
                plsc.addupdate_scatter(hist_k, [a], qvals)
            return 0
        lax.fori_loop(0, D // LANES, dbody, 0)

    def cbody(k, _):
        wait(k)
        compute(k)
        @pl.when(k < N_CHUNKS - 2)
        def _():
            start(k + 2)
        return 0
    lax.fori_loop(0, N_CHUNKS, cbody, 0)

    tr_idx = [((lane + j) & (LANES - 1)) * LANES + lane for j in range(LANES)]
    out_col = [(lane + j) & (LANES - 1) for j in range(LANES)]
    hi_mask = [((lane + j) & (LANES - 1)) <= (BINS - LANES - 1)
               for j in range(LANES)]

    def tbody(k, _):
        hist_k = hist.at[pl.ds(k * HIST_PER_CHUNK, HIST_PER_CHUNK)]
        pk = packed_v.at[k >> 1, pl.ds((k & 1) * CHUNK, CHUNK), :]
        for j in range(LANES):
            g = plsc.load_gather(hist_k, [tr_idx[j]])
            plsc.store_scatter(pk, [lane, out_col[j]], g)
        for j in range(LANES):
            g = plsc.load_gather(hist_k, [tr_idx[j] + LANES * LANES])
            plsc.store_scatter(pk, [lane, out_col[j] + LANES], g,
                               mask=hi_mask[j])
        return 0
    lax.fori_loop(0, N_CHUNKS, tbody, 0)

    pltpu.sync_copy(packed_v, out_hbm.at[wid])


@functools.cache
def _build_kernel():
    mesh = plsc.VectorSubcoreMesh(core_axis_name="c", subcore_axis_name="s")
    return pl.kernel(
        _hist_kernel_body,
        out_type=jax.ShapeDtypeStruct((B, C, Q, BINS), jnp.float32),
        mesh=mesh,
        compiler_params=pltpu.CompilerParams(needs_layout_passes=False),
        scratch_types=[
            pltpu.VMEM((2, CHUNK, D), jnp.float32),
            pltpu.VMEM((ROWS_PER_W * NBINS_PAD,), jnp.float32),
            pltpu.VMEM((C, Q, BINS), jnp.float32),
            pltpu.VMEM((D,), jnp.int32),
            pltpu.VMEM((Q,), jnp.int32),
            pltpu.VMEM((D,), jnp.float32),
            pltpu.VMEM((Q,), jnp.float32),
            pltpu.SemaphoreType.DMA((2,)),
        ],
    )


def kernel(simmat, dlens, dtoks, qtoks):
    del dlens
    sim2d = simmat.reshape(ROWS, D)
    return _build_kernel()(sim2d, dtoks.astype(jnp.int32),
                           qtoks.astype(jnp.int32))
